# baseline (speedup 1.0000x reference)
import functools

import jax
import jax.numpy as jnp
from jax import lax
from jax.experimental import pallas as pl
from jax.experimental.pallas import tpu as pltpu
from jax.experimental.pallas import tpu_sc as plsc

D_MODEL = 64
BI = 512
NBLK = (1000000 + BI - 1) // BI
SCALE = 8.0
NC, NS, L = 2, 16, 16
NW = NC * NS
TG, TR = 25, 8
SM, SR = 32, 128
JG, JR = 8, 8
NT = TG * TR
NG = 4
NI = 3
TW = 2 * D_MODEL
TV = 1000000


def _tc_repack(tT):
    def body(x_ref, y_ref):
        eye = jnp.eye(D_MODEL, dtype=jnp.float32)
        y_ref[:, 0:D_MODEL] = lax.dot_general(
            x_ref[...], eye, (((0,), (0,)), ((), ())),
            preferred_element_type=jnp.float32)

    return pl.pallas_call(
        body,
        grid=(NBLK,),
        in_specs=[pl.BlockSpec((D_MODEL, BI), lambda i: (0, i))],
        out_specs=pl.BlockSpec((BI, TW), lambda i: (i, 0)),
        out_shape=jax.ShapeDtypeStruct((TV, TW), jnp.float32),
    )(tT)


def _sc_embed(x2, t2):
    mesh = plsc.VectorSubcoreMesh(core_axis_name="c", subcore_axis_name="s")

    @functools.partial(
        pl.kernel,
        out_type=jax.ShapeDtypeStruct((NT, JG, SM, JR, SR), jnp.float32),
        mesh=mesh,
        scratch_types=(
            [pltpu.VMEM((NI, TR, SR), jnp.int32)]
            + [pltpu.VMEM((SR, TW), jnp.float32)
               for _ in range(NG)]
            + [pltpu.VMEM((D_MODEL, SR), jnp.float32)
               for _ in range(2)]
            + [pltpu.SemaphoreType.DMA] * NG
            + [pltpu.SemaphoreType.DMA] * 2
            + [pltpu.SemaphoreType.DMA]
        ),
        compiler_params=pltpu.CompilerParams(use_tc_tiling_on_sc=False,
                                             needs_layout_passes=False),
    )
    def body(x_hbm, table_hbm, out_hbm, ib, *scratch):
        G = scratch[0:NG]
        GT = scratch[NG:NG + 2]
        sg = scratch[NG + 2:2 * NG + 2]
        so = scratch[2 * NG + 2:2 * NG + 4]
        si = scratch[2 * NG + 4]

        wid = lax.axis_index("s") * NC + lax.axis_index("c")

        def wait_idx():
            pltpu.make_async_copy(x_hbm.at[0, 0], ib.at[0], si).wait()

        def fire_idx(blk):
            bc = jnp.minimum(blk, TG - 1)
            pltpu.async_copy(x_hbm.at[bc, wid], ib.at[lax.rem(blk, NI)], si)

        def fire_gather(blk, row, b):
            pltpu.async_copy(table_hbm.at[ib.at[lax.rem(blk, NI), row]],
                             G[b], sg[b])

        def wait_gather(b):
            pltpu.make_async_copy(table_hbm.at[pl.ds(0, SR)], G[b],
                                  sg[b]).wait()

        def wait_out(b2):
            pltpu.make_async_copy(table_hbm.at[pl.ds(0, D_MODEL)], GT[b2],
                                  so[b2]).wait()

        def transpose_scale(b, b2):
            src, dst = G[b], GT[b2]

            @plsc.parallel_loop(0, SR, L, unroll=2)
            def _tp(sr0):
                lane = lax.iota(jnp.int32, L)
                rows = sr0 + lane

                def jblk(jc, cc):
                    for jj in range(16):
                        cvj = (lane + jc * 16 + jj) & (D_MODEL - 1)
                        v = plsc.load_gather(src, [rows, cvj])
                        plsc.store_scatter(dst, [cvj, rows], v * SCALE)
                    return cc

                lax.fori_loop(0, D_MODEL // 16, jblk, 0)

        fire_idx(0)
        wait_idx()
        for tr in range(NG):
            fire_gather(0, tr, tr)
        fire_idx(1)

        def block(g, carry):
            for tr in range(TR):
                t = g * TR + tr
                b = tr % NG
                b2 = tr % 2
                if tr == 0:
                    wait_idx()
                if tr == 1:
                    fire_idx(g + 2)
                wait_gather(b)
                if tr >= 2:
                    wait_out(b2)
                else:
                    @pl.when(g > 0)
                    def _():
                        wait_out(b2)
                transpose_scale(b, b2)
                for jg in range(JG):
                    pltpu.async_copy(GT[b2].at[pl.ds(jg * JR, JR)],
                                     out_hbm.at[t, jg, wid], so[b2])

                if tr < TR - NG:
                    fire_gather(g, tr + NG, b)
                else:
                    @pl.when(g < TG - 1)
                    def _():
                        fire_gather(g + 1, tr + NG - TR, b)
            return carry

        lax.fori_loop(0, TG, block, 0)

        wait_out(0)
        wait_out(1)
        wait_idx()

    return body(x2, t2)


def kernel(x, table):
    x2 = x.T.reshape(TG, TR, SM, SR).transpose(0, 2, 1, 3).astype(jnp.int32)
    t2 = _tc_repack(table.T)
    out5 = _sc_embed(x2, t2)
    return out5.transpose(2, 4, 0, 1, 3).reshape(SM * SR, NT, JG * JR)


# speedup vs baseline: 1.5744x; 1.5744x over previous
import functools

import jax
import jax.numpy as jnp
from jax import lax
from jax.experimental import pallas as pl
from jax.experimental.pallas import tpu as pltpu
from jax.experimental.pallas import tpu_sc as plsc

D_MODEL = 64
SCALE = 8.0
NC, NS, L = 2, 16, 16
NW = NC * NS
TG, TR = 25, 8
SM, SR = 32, 128
JG, JR = 8, 8
NT = TG * TR
NG = 4
NI = 3
TW = 2 * D_MODEL
TV = 500000


def _sc_embed(x2, t2):
    mesh = plsc.VectorSubcoreMesh(core_axis_name="c", subcore_axis_name="s")

    @functools.partial(
        pl.kernel,
        out_type=jax.ShapeDtypeStruct((NT, JG, SM, JR, SR), jnp.float32),
        mesh=mesh,
        scratch_types=(
            [pltpu.VMEM((NI, TR, SR), jnp.int32)]
            + [pltpu.VMEM((2, TR, SR), jnp.int32)] * 2
            + [pltpu.VMEM((SR, TW), jnp.float32)
               for _ in range(NG)]
            + [pltpu.VMEM((D_MODEL, SR), jnp.float32)
               for _ in range(2)]
            + [pltpu.SemaphoreType.DMA] * NG
            + [pltpu.SemaphoreType.DMA] * 2
            + [pltpu.SemaphoreType.DMA]
        ),
        compiler_params=pltpu.CompilerParams(use_tc_tiling_on_sc=False,
                                             needs_layout_passes=False),
    )
    def body(x_hbm, table_hbm, out_hbm, ib, ibh, ibc, *scratch):
        G = scratch[0:NG]
        GT = scratch[NG:NG + 2]
        sg = scratch[NG + 2:2 * NG + 2]
        so = scratch[2 * NG + 2:2 * NG + 4]
        si = scratch[2 * NG + 4]

        wid = lax.axis_index("s") * NC + lax.axis_index("c")

        def wait_idx():
            pltpu.make_async_copy(x_hbm.at[0, 0], ib.at[0], si).wait()

        def fire_idx(blk):
            bc = jnp.minimum(blk, TG - 1)
            pltpu.async_copy(x_hbm.at[bc, wid], ib.at[lax.rem(blk, NI)], si)

        def derive_idx(blk):
            rs = lax.rem(blk, NI)
            ds2 = lax.rem(blk, 2)
            for r in range(TR):
                for c in range(SR // L):
                    v = ib[rs, r, pl.ds(c * L, L)]
                    ibh[ds2, r, pl.ds(c * L, L)] = v >> 1
                    ibc[ds2, r, pl.ds(c * L, L)] = (v & 1) << 6

        def fire_gather(blk, row, b):
            pltpu.async_copy(table_hbm.at[ibh.at[lax.rem(blk, 2), row]],
                             G[b], sg[b])

        def wait_gather(b):
            pltpu.make_async_copy(table_hbm.at[pl.ds(0, SR)], G[b],
                                  sg[b]).wait()

        def wait_out(b2):
            pltpu.make_async_copy(table_hbm.at[pl.ds(0, D_MODEL)], GT[b2],
                                  so[b2]).wait()

        def transpose_scale(blk, row, b, b2):
            src, dst = G[b], GT[b2]
            ds2 = lax.rem(blk, 2)

            @plsc.parallel_loop(0, SR, L, unroll=2)
            def _tp(sr0):
                lane = lax.iota(jnp.int32, L)
                rows = sr0 + lane
                cofv = ibc[ds2, row, pl.ds(sr0, L)]

                def jblk(jc, cc):
                    for jj in range(16):
                        cvj = (lane + jc * 16 + jj) & (D_MODEL - 1)
                        v = plsc.load_gather(src, [rows, cofv + cvj])
                        plsc.store_scatter(dst, [cvj, rows], v * SCALE)
                    return cc

                lax.fori_loop(0, D_MODEL // 16, jblk, 0)

        fire_idx(0)
        wait_idx()
        derive_idx(0)
        for tr in range(NG):
            fire_gather(0, tr, tr)
        fire_idx(1)

        def block(g, carry):
            for tr in range(TR):
                t = g * TR + tr
                b = tr % NG
                b2 = tr % 2
                if tr == 0:
                    wait_idx()
                    derive_idx(g + 1)
                if tr == 1:
                    fire_idx(g + 2)
                wait_gather(b)
                if tr >= 2:
                    wait_out(b2)
                else:
                    @pl.when(g > 0)
                    def _():
                        wait_out(b2)
                transpose_scale(g, tr, b, b2)
                for jg in range(JG):
                    pltpu.async_copy(GT[b2].at[pl.ds(jg * JR, JR)],
                                     out_hbm.at[t, jg, wid], so[b2])

                if tr < TR - NG:
                    fire_gather(g, tr + NG, b)
                else:
                    @pl.when(g < TG - 1)
                    def _():
                        fire_gather(g + 1, tr + NG - TR, b)
            return carry

        lax.fori_loop(0, TG, block, 0)

        wait_out(0)
        wait_out(1)
        wait_idx()

    return body(x2, t2)


def kernel(x, table):
    x2 = x.T.reshape(TG, TR, SM, SR).transpose(0, 2, 1, 3).astype(jnp.int32)
    t2 = table.reshape(TV, TW)
    out5 = _sc_embed(x2, t2)
    return out5.transpose(2, 4, 0, 1, 3).reshape(SM * SR, NT, JG * JR)
